# baseline (speedup 1.0000x reference)
import functools

import jax
import jax.numpy as jnp
from jax import lax
from jax.experimental import pallas as pl
from jax.experimental.pallas import tpu as pltpu
from jax.experimental.pallas import tpu_sc as plsc

N = 10000
E = 320000
D = 128
G = 64

NC = 2
NS = 16
L = 16
NW = NC * NS
CHUNK = 128
EW = E // NW
TRIPLE = -(-EW // (3 * CHUNK))
NCH = 3 * TRIPLE
ETP = NCH * CHUNK
EP = NW * ETP
N_PAD = N
STRIPE = 632
STRIPE_LAST = N_PAD - 15 * STRIPE

_mesh = plsc.VectorSubcoreMesh(core_axis_name="c", subcore_axis_name="s")
_sc_params = pltpu.CompilerParams(needs_layout_passes=False)


@functools.partial(
    pl.kernel,
    out_type=jax.ShapeDtypeStruct((NW, N_PAD), jnp.float32),
    mesh=_mesh,
    compiler_params=_sc_params,
    scratch_types=[
        pltpu.VMEM((ETP,), jnp.int32),
        pltpu.VMEM((ETP,), jnp.float32),
        pltpu.VMEM((N_PAD,), jnp.float32),
    ],
)
def _deg_kernel(col_hbm, ew_hbm, deg_out, col_v, ew_v, deg_v):
    wid = lax.axis_index("s") * NC + lax.axis_index("c")
    pltpu.sync_copy(col_hbm.at[wid], col_v)
    pltpu.sync_copy(ew_hbm.at[wid], ew_v)
    zeros = jnp.zeros((L,), jnp.float32)

    def zbody(i, carry):
        deg_v[pl.ds(pl.multiple_of(i * L, L), L)] = zeros
        return carry

    lax.fori_loop(0, N_PAD // L, zbody, 0)

    def ebody(i, carry):
        sl = pl.ds(pl.multiple_of(i * L, L), L)
        plsc.addupdate_scatter(deg_v, [col_v[sl]], ew_v[sl])
        return carry

    lax.fori_loop(0, ETP // L, ebody, 0)
    pltpu.sync_copy(deg_v, deg_out.at[wid])


def _prep_body(x_ref, w_ref, degp_ref, hp_ref, dis_ref):
    deg = jnp.sum(degp_ref[...], axis=0) + 1.0
    dis = jnp.where(deg > 0, lax.rsqrt(deg), 0.0)
    h = jnp.dot(x_ref[...], w_ref[...], preferred_element_type=jnp.float32)
    hp_ref[...] = h * dis[:, None]
    dis_ref[...] = dis


_prep_call = pl.pallas_call(
    _prep_body,
    out_shape=(
        jax.ShapeDtypeStruct((N, D), jnp.float32),
        jax.ShapeDtypeStruct((N,), jnp.float32),
    ),
)


@functools.partial(
    pl.kernel,
    out_type=jax.ShapeDtypeStruct((NC, N_PAD, D), jnp.float32),
    mesh=_mesh,
    compiler_params=_sc_params,
    scratch_types=[
        pltpu.VMEM((3, CHUNK), jnp.int32),
        pltpu.VMEM((3, CHUNK), jnp.int32),
        pltpu.VMEM((3, CHUNK), jnp.float32),
        pltpu.VMEM((CHUNK, D), jnp.float32),
        pltpu.VMEM((CHUNK, D), jnp.float32),
        pltpu.VMEM((CHUNK, D), jnp.float32),
        pltpu.VMEM_SHARED((N_PAD, D), jnp.float32),
        pltpu.SemaphoreType.DMA,
        pltpu.SemaphoreType.DMA,
        pltpu.SemaphoreType.DMA,
        pltpu.SemaphoreType.DMA,
        pltpu.SemaphoreType.DMA,
        pltpu.SemaphoreType.DMA,
        pltpu.SemaphoreType.DMA,
        pltpu.SemaphoreType.DMA,
        pltpu.SemaphoreType.DMA,
        pltpu.SemaphoreType.DMA,
        pltpu.SemaphoreType.DMA,
        pltpu.SemaphoreType.DMA,
    ],
)
def _msg_kernel(row_hbm, col_hbm, ew_hbm, hp_hbm, out_hbm,
                rowr, colr, ewr, b0, b1, b2, acc,
                gs0, gs1, gs2, ss0, ss1, ss2,
                rs0, rs1, rs2, cs0, cs1, cs2):
    cid = lax.axis_index("c")
    sid = lax.axis_index("s")
    wid = sid * NC + cid
    bufs = (b0, b1, b2)
    gs = (gs0, gs1, gs2)
    ss = (ss0, ss1, ss2)
    rs = (rs0, rs1, rs2)
    cs = (cs0, cs1, cs2)

    def _gwait(sem):
        pltpu.make_async_copy(hp_hbm.at[pl.ds(0, CHUNK)], b0, sem).wait()
        pltpu.make_async_copy(ew_hbm.at[0].at[0], ewr.at[0], sem).wait()

    def _swait(sem):
        pltpu.make_async_copy(hp_hbm.at[pl.ds(0, CHUNK)], b0, sem).wait()

    def _rwait(sem):
        pltpu.make_async_copy(row_hbm.at[0].at[0], rowr.at[0], sem).wait()

    def _cwait(sem):
        pltpu.make_async_copy(col_hbm.at[0].at[0], colr.at[0], sem).wait()

    for c in range(3):
        pltpu.sync_copy(row_hbm.at[wid].at[c], rowr.at[c])
        pltpu.sync_copy(col_hbm.at[wid].at[c], colr.at[c])
    for c in (1, 2):
        pltpu.async_copy(hp_hbm.at[rowr.at[c]], bufs[c], gs[c])
        pltpu.async_copy(ew_hbm.at[wid].at[c], ewr.at[c], gs[c])

    zeros = jnp.zeros((L,), jnp.float32)

    def zbody(i, carry):
        r = i // (D // L)
        col = (i % (D // L)) * L
        b0[r, pl.ds(col, L)] = zeros
        return carry

    lax.fori_loop(0, CHUNK * D // L, zbody, 0)
    base = pl.multiple_of(sid * STRIPE, 8)
    for k in range(4):
        pltpu.sync_copy(b0, acc.at[pl.ds(base + k * CHUNK, CHUNK)])

    @pl.when(sid < NS - 1)
    def _():
        pltpu.sync_copy(b0.at[pl.ds(0, STRIPE - 4 * CHUNK)],
                        acc.at[pl.ds(base + 4 * CHUNK, STRIPE - 4 * CHUNK)])

    @pl.when(sid == NS - 1)
    def _():
        pltpu.sync_copy(
            b0.at[pl.ds(0, STRIPE_LAST - 4 * CHUNK)],
            acc.at[pl.ds(base + 4 * CHUNK, STRIPE_LAST - 4 * CHUNK)])

    pltpu.async_copy(hp_hbm.at[rowr.at[0]], b0, gs0)
    pltpu.async_copy(ew_hbm.at[wid].at[0], ewr.at[0], gs0)
    plsc.subcore_barrier()

    bvecs = tuple(jnp.full((L,), c, jnp.int32) for c in range(3))

    def _scale(b):
        def rbody(r, carry):
            s = plsc.load_gather(ewr, [bvecs[b], jnp.full((L,), r, jnp.int32)])
            for j in range(D // L):
                sl = pl.ds(j * L, L)
                bufs[b][r, sl] = bufs[b][r, sl] * s
            return carry

        lax.fori_loop(0, CHUNK, rbody, 0)

    def turn_body(i, carry):
        for c in range(3):
            ch = 3 * i + c
            b = c
            bprev = (c + 2) % 3
            _gwait(gs[b])

            @pl.when(i < TRIPLE - 1)
            def _():
                pltpu.async_copy(row_hbm.at[wid].at[ch + 3], rowr.at[b], rs[b])

            @pl.when(i > 0)
            def _():
                _cwait(cs[b])
            pltpu.async_copy(bufs[b], acc.at[colr.at[b]], ss[b], add=True)

            def _tail():
                _swait(ss[bprev])

                @pl.when(ch + 2 < NCH)
                def _():
                    pltpu.async_copy(col_hbm.at[wid].at[ch + 2],
                                     colr.at[bprev], cs[bprev])
                    _rwait(rs[bprev])
                    pltpu.async_copy(hp_hbm.at[rowr.at[bprev]],
                                     bufs[bprev], gs[bprev])
                    pltpu.async_copy(ew_hbm.at[wid].at[ch + 2],
                                     ewr.at[bprev], gs[bprev])

            if c == 0:
                @pl.when(i > 0)
                def _():
                    _tail()
            else:
                _tail()
        return carry

    lax.fori_loop(0, TRIPLE, turn_body, 0)
    _swait(ss[(NCH - 1) % 3])
    plsc.subcore_barrier()
    dbase = pl.multiple_of(sid * STRIPE, 8)

    @pl.when(sid < NS - 1)
    def _():
        pltpu.sync_copy(acc.at[pl.ds(dbase, STRIPE)],
                        out_hbm.at[cid].at[pl.ds(dbase, STRIPE)])

    @pl.when(sid == NS - 1)
    def _():
        pltpu.sync_copy(acc.at[pl.ds(dbase, STRIPE_LAST)],
                        out_hbm.at[cid].at[pl.ds(dbase, STRIPE_LAST)])


def _post_body(sp_ref, hp_ref, dis_ref, bconv_ref, batch_ref,
               gnw_ref, gnb_ref, gms_ref, y_ref):
    s = sp_ref[0] + sp_ref[1]
    dis = dis_ref[...]
    out = dis[:, None] * (s + hp_ref[...]) + bconv_ref[...]

    batch = batch_ref[...]
    gids = lax.iota(jnp.int32, G)
    oh_ng = (batch[:, None] == gids[None, :]).astype(jnp.float32)
    cnt = jnp.maximum(jnp.sum(oh_ng, axis=0), 1.0)
    sums = lax.dot_general(oh_ng, out, (((0,), (0,)), ((), ())),
                           preferred_element_type=jnp.float32)
    sumsq = lax.dot_general(oh_ng, out * out, (((0,), (0,)), ((), ())),
                            preferred_element_type=jnp.float32)
    mean = sums / cnt[:, None]
    m2 = sumsq / cnt[:, None]
    gms = gms_ref[...]
    var = m2 + (gms * gms - 2.0 * gms) * (mean * mean)
    inv_std = lax.rsqrt(var + 1e-5)
    mean_row = jnp.dot(oh_ng, mean, preferred_element_type=jnp.float32)
    isd_row = jnp.dot(oh_ng, inv_std, preferred_element_type=jnp.float32)
    out_c = out - mean_row * gms
    y = gnw_ref[...] * out_c * isd_row + gnb_ref[...]
    y_ref[...] = jnp.where(y > 0, y, 0.1 * y)


_post_call = pl.pallas_call(
    _post_body,
    out_shape=jax.ShapeDtypeStruct((N, D), jnp.float32),
)


def kernel(x, edge_index, edge_weight, batch, W, b_conv, gn_weight, gn_bias,
           gn_mean_scale):
    row = edge_index[0].astype(jnp.int32)
    col = edge_index[1].astype(jnp.int32)
    batch32 = batch.astype(jnp.int32)
    pad = EP - E
    rowp = jnp.concatenate([row, jnp.zeros((pad,), jnp.int32)])
    colp = jnp.concatenate([col, jnp.zeros((pad,), jnp.int32)])
    ewp = jnp.concatenate([edge_weight.astype(jnp.float32),
                           jnp.zeros((pad,), jnp.float32)])
    row3 = rowp.reshape(NW, NCH, CHUNK)
    col3 = colp.reshape(NW, NCH, CHUNK)
    ew3 = ewp.reshape(NW, NCH, CHUNK)
    colf = colp.reshape(NW, ETP)
    ewf = ewp.reshape(NW, ETP)

    degp = _deg_kernel(colf, ewf)
    hp, dis = _prep_call(x, W, degp)
    spart = _msg_kernel(row3, col3, ew3, hp)
    y = _post_call(spart, hp, dis, b_conv, batch32, gn_weight, gn_bias,
                   gn_mean_scale)
    return y


# speedup vs baseline: 1.2380x; 1.2380x over previous
import functools

import jax
import jax.numpy as jnp
from jax import lax
from jax.experimental import pallas as pl
from jax.experimental.pallas import tpu as pltpu
from jax.experimental.pallas import tpu_sc as plsc

N = 10000
E = 320000
D = 128
G = 64

NC = 2
NS = 16
L = 16
NW = NC * NS
CHUNK = 128
EW = E // NW
PAIRS = -(-EW // (2 * CHUNK))
NCH = 2 * PAIRS
ETP = NCH * CHUNK
EP = NW * ETP
N_PAD = N
STRIPE = 632
STRIPE_LAST = N_PAD - 15 * STRIPE

_mesh = plsc.VectorSubcoreMesh(core_axis_name="c", subcore_axis_name="s")
_sc_params = pltpu.CompilerParams(needs_layout_passes=False)


@functools.partial(
    pl.kernel,
    out_type=jax.ShapeDtypeStruct((NW, N_PAD), jnp.float32),
    mesh=_mesh,
    compiler_params=_sc_params,
    scratch_types=[
        pltpu.VMEM((ETP,), jnp.int32),
        pltpu.VMEM((ETP,), jnp.float32),
        pltpu.VMEM((N_PAD,), jnp.float32),
    ],
)
def _deg_kernel(col_hbm, ew_hbm, deg_out, col_v, ew_v, deg_v):
    wid = lax.axis_index("s") * NC + lax.axis_index("c")
    pltpu.sync_copy(col_hbm.at[wid], col_v)
    pltpu.sync_copy(ew_hbm.at[wid], ew_v)
    zeros = jnp.zeros((L,), jnp.float32)

    def zbody(i, carry):
        deg_v[pl.ds(pl.multiple_of(i * L, L), L)] = zeros
        return carry

    lax.fori_loop(0, N_PAD // L, zbody, 0)

    def ebody(i, carry):
        sl = pl.ds(pl.multiple_of(i * L, L), L)
        plsc.addupdate_scatter(deg_v, [col_v[sl]], ew_v[sl])
        return carry

    lax.fori_loop(0, ETP // L, ebody, 0)
    pltpu.sync_copy(deg_v, deg_out.at[wid])


def _prep_body(x_ref, w_ref, degp_ref, hp_ref, dis_ref):
    deg = jnp.sum(degp_ref[...], axis=0) + 1.0
    dis = jnp.where(deg > 0, lax.rsqrt(deg), 0.0)
    h = jnp.dot(x_ref[...], w_ref[...], preferred_element_type=jnp.float32)
    hp_ref[...] = h * dis[:, None]
    dis_ref[...] = dis


_prep_call = pl.pallas_call(
    _prep_body,
    out_shape=(
        jax.ShapeDtypeStruct((N, D), jnp.float32),
        jax.ShapeDtypeStruct((N,), jnp.float32),
    ),
)


@functools.partial(
    pl.kernel,
    out_type=jax.ShapeDtypeStruct((NC, N_PAD, D), jnp.float32),
    mesh=_mesh,
    compiler_params=_sc_params,
    scratch_types=[
        pltpu.VMEM((ETP,), jnp.int32),
        pltpu.VMEM((ETP // 2,), jnp.int32),
        pltpu.VMEM((2, CHUNK), jnp.int32),
        pltpu.VMEM((2, CHUNK), jnp.int32),
        pltpu.VMEM((CHUNK, D), jnp.float32),
        pltpu.VMEM((CHUNK, D), jnp.float32),
        pltpu.VMEM_SHARED((N_PAD, D), jnp.float32),
        pltpu.SemaphoreType.DMA,
        pltpu.SemaphoreType.DMA,
        pltpu.SemaphoreType.DMA,
        pltpu.SemaphoreType.DMA,
    ],
)
def _msg_kernel(rc_hbm, ewq_hbm, hp_hbm, out_hbm,
                rc_v, ewq_v, rowr, colr, b0, b1, acc,
                gs0, gs1, ss0, ss1):
    cid = lax.axis_index("c")
    sid = lax.axis_index("s")
    wid = sid * NC + cid
    bufs = (b0, b1)
    gs = (gs0, gs1)
    ss = (ss0, ss1)

    def _gwait(sem):
        pltpu.make_async_copy(hp_hbm.at[pl.ds(0, CHUNK)], b0, sem).wait()

    _swait = _gwait

    pltpu.sync_copy(rc_hbm.at[wid], rc_v)
    pltpu.sync_copy(ewq_hbm.at[wid], ewq_v)

    lo16 = jnp.full((L,), 0xFFFF, jnp.int32)

    def _unpack_row(ch, b):
        base = pl.multiple_of(ch * CHUNK, CHUNK)
        for g in range(CHUNK // L):
            v = rc_v[pl.ds(base + g * L, L)]
            rowr[b, pl.ds(g * L, L)] = v & lo16

    def _unpack_col(ch, b):
        base = pl.multiple_of(ch * CHUNK, CHUNK)
        for g in range(CHUNK // L):
            v = rc_v[pl.ds(base + g * L, L)]
            colr[b, pl.ds(g * L, L)] = lax.shift_right_logical(v, 16)

    _unpack_row(0, 0)
    _unpack_row(1, 1)
    _unpack_col(0, 0)
    _unpack_col(1, 1)

    zeros = jnp.zeros((L,), jnp.float32)

    def zbody(i, carry):
        r = i // (D // L)
        col = (i % (D // L)) * L
        b0[r, pl.ds(col, L)] = zeros
        return carry

    lax.fori_loop(0, CHUNK * D // L, zbody, 0)
    zbase = pl.multiple_of(sid * STRIPE, 8)
    for k in range(4):
        pltpu.sync_copy(b0, acc.at[pl.ds(zbase + k * CHUNK, CHUNK)])

    @pl.when(sid < NS - 1)
    def _():
        pltpu.sync_copy(b0.at[pl.ds(0, STRIPE - 4 * CHUNK)],
                        acc.at[pl.ds(zbase + 4 * CHUNK, STRIPE - 4 * CHUNK)])

    @pl.when(sid == NS - 1)
    def _():
        pltpu.sync_copy(
            b0.at[pl.ds(0, STRIPE_LAST - 4 * CHUNK)],
            acc.at[pl.ds(zbase + 4 * CHUNK, STRIPE_LAST - 4 * CHUNK)])

    pltpu.async_copy(hp_hbm.at[rowr.at[0]], b0, gs0)
    pltpu.async_copy(hp_hbm.at[rowr.at[1]], b1, gs1)
    plsc.subcore_barrier()

    qscale = jnp.full((L,), 1.0 / 65535.0, jnp.float32)

    def _scale(ch, b):
        base2 = ch * (CHUNK // 2)

        def rbody(r, carry):
            v = plsc.load_gather(
                ewq_v,
                [jnp.full((L,), base2, jnp.int32)
                 + lax.shift_right_logical(jnp.full((L,), r, jnp.int32), 1)])
            sh = (r & 1) * 16
            q = lax.shift_right_logical(v, jnp.full((L,), sh, jnp.int32))
            s = (q & lo16).astype(jnp.float32) * qscale
            for j in range(D // L):
                sl = pl.ds(j * L, L)
                bufs[b][r, sl] = bufs[b][r, sl] * s
            return carry

        lax.fori_loop(0, CHUNK, rbody, 0)

    def pair_body(i, carry):
        for c in range(2):
            ch = 2 * i + c
            b = c
            bo = 1 - c
            _gwait(gs[b])

            @pl.when(i < PAIRS - 1)
            def _():
                _unpack_row(ch + 2, b)

            def _mid():
                _swait(ss[bo])

            def _mid2():
                pltpu.async_copy(hp_hbm.at[rowr.at[bo]], bufs[bo], gs[bo])
                _unpack_col(ch + 1, bo)

            if c == 0:
                @pl.when(i > 0)
                def _():
                    _mid()

                @pl.when(i > 0)
                def _():
                    _mid2()
            else:
                _mid()

                @pl.when(i < PAIRS - 1)
                def _():
                    _mid2()

            _scale(ch, b)
            pltpu.async_copy(bufs[b], acc.at[colr.at[b]], ss[b], add=True)
        return carry

    lax.fori_loop(0, PAIRS, pair_body, 0)
    _swait(ss[(NCH - 1) % 2])
    plsc.subcore_barrier()
    dbase = pl.multiple_of(sid * STRIPE, 8)

    @pl.when(sid < NS - 1)
    def _():
        pltpu.sync_copy(acc.at[pl.ds(dbase, STRIPE)],
                        out_hbm.at[cid].at[pl.ds(dbase, STRIPE)])

    @pl.when(sid == NS - 1)
    def _():
        pltpu.sync_copy(acc.at[pl.ds(dbase, STRIPE_LAST)],
                        out_hbm.at[cid].at[pl.ds(dbase, STRIPE_LAST)])


def _post_body(sp_ref, hp_ref, dis_ref, bconv_ref, batch_ref,
               gnw_ref, gnb_ref, gms_ref, y_ref):
    s = sp_ref[0] + sp_ref[1]
    dis = dis_ref[...]
    out = dis[:, None] * (s + hp_ref[...]) + bconv_ref[...]

    batch = batch_ref[...]
    gids = lax.iota(jnp.int32, G)
    oh_ng = (batch[:, None] == gids[None, :]).astype(jnp.float32)
    cnt = jnp.maximum(jnp.sum(oh_ng, axis=0), 1.0)
    sums = lax.dot_general(oh_ng, out, (((0,), (0,)), ((), ())),
                           preferred_element_type=jnp.float32)
    sumsq = lax.dot_general(oh_ng, out * out, (((0,), (0,)), ((), ())),
                            preferred_element_type=jnp.float32)
    mean = sums / cnt[:, None]
    m2 = sumsq / cnt[:, None]
    gms = gms_ref[...]
    var = m2 + (gms * gms - 2.0 * gms) * (mean * mean)
    inv_std = lax.rsqrt(var + 1e-5)
    mean_row = jnp.dot(oh_ng, mean, preferred_element_type=jnp.float32)
    isd_row = jnp.dot(oh_ng, inv_std, preferred_element_type=jnp.float32)
    out_c = out - mean_row * gms
    y = gnw_ref[...] * out_c * isd_row + gnb_ref[...]
    y_ref[...] = jnp.where(y > 0, y, 0.1 * y)


_post_call = pl.pallas_call(
    _post_body,
    out_shape=jax.ShapeDtypeStruct((N, D), jnp.float32),
)


def kernel(x, edge_index, edge_weight, batch, W, b_conv, gn_weight, gn_bias,
           gn_mean_scale):
    row = edge_index[0].astype(jnp.int32)
    col = edge_index[1].astype(jnp.int32)
    batch32 = batch.astype(jnp.int32)
    pad = EP - E
    rowp = jnp.concatenate([row, jnp.zeros((pad,), jnp.int32)])
    colp = jnp.concatenate([col, jnp.zeros((pad,), jnp.int32)])
    ewp = jnp.concatenate([edge_weight.astype(jnp.float32),
                           jnp.zeros((pad,), jnp.float32)])
    rc = (rowp | (colp << 16)).reshape(NW, ETP)
    q = jnp.round(ewp * 65535.0).astype(jnp.int32)
    ewq = (q[0::2] | (q[1::2] << 16)).reshape(NW, ETP // 2)
    colf = colp.reshape(NW, ETP)
    ewf = ewp.reshape(NW, ETP)

    degp = _deg_kernel(colf, ewf)
    hp, dis = _prep_call(x, W, degp)
    spart = _msg_kernel(rc, ewq, hp)
    y = _post_call(spart, hp, dis, b_conv, batch32, gn_weight, gn_bias,
                   gn_mean_scale)
    return y


# speedup vs baseline: 1.8071x; 1.4597x over previous
import functools

import jax
import jax.numpy as jnp
from jax import lax
from jax.experimental import pallas as pl
from jax.experimental.pallas import tpu as pltpu
from jax.experimental.pallas import tpu_sc as plsc

N = 10000
E = 320000
D = 128
G = 64

NC = 2
NS = 16
L = 16
NW = NC * NS
CHUNK = 64
ET = E // NW
NCH = 2 * (-(-ET // (2 * CHUNK)))
NPAIR = NCH // 2
ETP = NCH * CHUNK
EP = NW * ETP
N_PAD = 10240
STRIPE = N_PAD // NS

_mesh = plsc.VectorSubcoreMesh(core_axis_name="c", subcore_axis_name="s")
_sc_params = pltpu.CompilerParams(needs_layout_passes=False)


@functools.partial(
    pl.kernel,
    out_type=jax.ShapeDtypeStruct((NW, N_PAD), jnp.float32),
    mesh=_mesh,
    compiler_params=_sc_params,
    scratch_types=[
        pltpu.VMEM((ETP,), jnp.int32),
        pltpu.VMEM((ETP,), jnp.float32),
        pltpu.VMEM((N_PAD,), jnp.float32),
    ],
)
def _deg_kernel(col_hbm, ew_hbm, deg_out, col_v, ew_v, deg_v):
    wid = lax.axis_index("s") * NC + lax.axis_index("c")
    pltpu.sync_copy(col_hbm.at[wid], col_v)
    pltpu.sync_copy(ew_hbm.at[wid], ew_v)
    zeros = jnp.zeros((L,), jnp.float32)

    def zbody(i, carry):
        deg_v[pl.ds(pl.multiple_of(i * L, L), L)] = zeros
        return carry

    lax.fori_loop(0, N_PAD // L, zbody, 0)

    def ebody(i, carry):
        off = pl.ds(pl.multiple_of(i * L, L), L)
        plsc.addupdate_scatter(deg_v, [col_v[off]], ew_v[off])
        return carry

    lax.fori_loop(0, ETP // L, ebody, 0)
    pltpu.sync_copy(deg_v, deg_out.at[wid])


def _prep_body(x_ref, w_ref, degp_ref, hp_ref, dis_ref):
    deg = jnp.sum(degp_ref[...], axis=0)[:N] + 1.0
    dis = jnp.where(deg > 0, lax.rsqrt(deg), 0.0)
    h = jnp.dot(x_ref[...], w_ref[...], preferred_element_type=jnp.float32)
    hp_ref[...] = h * dis[:, None]
    dis_ref[...] = dis


_prep_call = pl.pallas_call(
    _prep_body,
    out_shape=(
        jax.ShapeDtypeStruct((N, D), jnp.float32),
        jax.ShapeDtypeStruct((N,), jnp.float32),
    ),
)


@functools.partial(
    pl.kernel,
    out_type=jax.ShapeDtypeStruct((NC, N_PAD, D), jnp.float32),
    mesh=_mesh,
    compiler_params=_sc_params,
    scratch_types=[
        pltpu.VMEM((ETP,), jnp.int32),
        pltpu.VMEM((ETP,), jnp.int32),
        pltpu.VMEM((ETP,), jnp.float32),
        pltpu.VMEM((CHUNK, D), jnp.float32),
        pltpu.VMEM((CHUNK, D), jnp.float32),
        pltpu.VMEM_SHARED((N_PAD, D), jnp.float32),
        pltpu.SemaphoreType.DMA,
        pltpu.SemaphoreType.DMA,
        pltpu.SemaphoreType.DMA,
        pltpu.SemaphoreType.DMA,
    ],
)
def _msg_kernel(row_hbm, col_hbm, ew_hbm, hp_hbm, out_hbm,
                row_v, col_v, ew_v, g0, g1, acc,
                gs0, gs1, ss0, ss1):
    cid = lax.axis_index("c")
    sid = lax.axis_index("s")
    wid = sid * NC + cid
    pltpu.sync_copy(row_hbm.at[wid], row_v)
    pltpu.sync_copy(col_hbm.at[wid], col_v)
    pltpu.sync_copy(ew_hbm.at[wid], ew_v)

    def _wait(sem):
        pltpu.make_async_copy(hp_hbm.at[pl.ds(0, CHUNK)], g0, sem).wait()

    zeros = jnp.zeros((L,), jnp.float32)

    def zbody(i, carry):
        r = i // (D // L)
        c = (i % (D // L)) * L
        g0[r, pl.ds(c, L)] = zeros
        return carry

    lax.fori_loop(0, CHUNK * D // L, zbody, 0)
    for k in range(STRIPE // CHUNK):
        pltpu.sync_copy(g0, acc.at[pl.ds(sid * STRIPE + k * CHUNK, CHUNK)])
    plsc.subcore_barrier()

    pltpu.async_copy(hp_hbm.at[row_v.at[pl.ds(0, CHUNK)]], g0, gs0)
    pltpu.async_copy(hp_hbm.at[row_v.at[pl.ds(CHUNK, CHUNK)]], g1, gs1)

    def _scale(ch, buf):
        base = ch * CHUNK

        def rbody(r, carry):
            s = plsc.load_gather(ew_v, [jnp.full((L,), base + r, jnp.int32)])
            for j in range(D // L):
                sl = pl.ds(j * L, L)
                buf[r, sl] = buf[r, sl] * s
            return carry

        lax.fori_loop(0, CHUNK, rbody, 0)

    def pair_body(i, carry):
        for (ch, buf, gsem, ssem) in (
            (2 * i, g0, gs0, ss0),
            (2 * i + 1, g1, gs1, ss1),
        ):
            _wait(gsem)
            _scale(ch, buf)
            cs = pl.ds(pl.multiple_of(ch * CHUNK, CHUNK), CHUNK)
            pltpu.async_copy(buf, acc.at[col_v.at[cs]], ssem, add=True)

            @pl.when(i < NPAIR - 1)
            def _():
                _wait(ssem)
                ns = pl.ds(pl.multiple_of((ch + 2) * CHUNK, CHUNK), CHUNK)
                pltpu.async_copy(hp_hbm.at[row_v.at[ns]], buf, gsem)
        return carry

    lax.fori_loop(0, NPAIR, pair_body, 0)
    _wait(ss0)
    _wait(ss1)
    plsc.subcore_barrier()
    pltpu.sync_copy(
        acc.at[pl.ds(sid * STRIPE, STRIPE)],
        out_hbm.at[cid, pl.ds(sid * STRIPE, STRIPE)],
    )


def _post_body(sp_ref, hp_ref, dis_ref, bconv_ref, batch_ref,
               gnw_ref, gnb_ref, gms_ref, y_ref):
    s = sp_ref[0, :N, :] + sp_ref[1, :N, :]
    dis = dis_ref[...]
    out = dis[:, None] * (s + hp_ref[...]) + bconv_ref[...]

    batch = batch_ref[...]
    gids = lax.iota(jnp.int32, G)
    oh_ng = (batch[:, None] == gids[None, :]).astype(jnp.float32)
    cnt = jnp.maximum(jnp.sum(oh_ng, axis=0), 1.0)
    sums = lax.dot_general(oh_ng, out, (((0,), (0,)), ((), ())),
                           preferred_element_type=jnp.float32)
    sumsq = lax.dot_general(oh_ng, out * out, (((0,), (0,)), ((), ())),
                            preferred_element_type=jnp.float32)
    mean = sums / cnt[:, None]
    m2 = sumsq / cnt[:, None]
    gms = gms_ref[...]
    var = m2 + (gms * gms - 2.0 * gms) * (mean * mean)
    inv_std = lax.rsqrt(var + 1e-5)
    mean_row = jnp.dot(oh_ng, mean, preferred_element_type=jnp.float32)
    isd_row = jnp.dot(oh_ng, inv_std, preferred_element_type=jnp.float32)
    out_c = out - mean_row * gms
    y = gnw_ref[...] * out_c * isd_row + gnb_ref[...]
    y_ref[...] = jnp.where(y > 0, y, 0.1 * y)


_post_call = pl.pallas_call(
    _post_body,
    out_shape=jax.ShapeDtypeStruct((N, D), jnp.float32),
)


def kernel(x, edge_index, edge_weight, batch, W, b_conv, gn_weight, gn_bias,
           gn_mean_scale):
    row = edge_index[0].astype(jnp.int32)
    col = edge_index[1].astype(jnp.int32)
    batch32 = batch.astype(jnp.int32)
    pad = EP - E
    rowp = jnp.concatenate([row, jnp.zeros((pad,), jnp.int32)])
    colp = jnp.concatenate([col, jnp.zeros((pad,), jnp.int32)])
    ewp = jnp.concatenate([edge_weight.astype(jnp.float32),
                           jnp.zeros((pad,), jnp.float32)])
    row2 = rowp.reshape(NW, ETP)
    col2 = colp.reshape(NW, ETP)
    colf = colp.reshape(NW, ETP)
    ewf = ewp.reshape(NW, ETP)

    degp = _deg_kernel(colf, ewf)
    hp, dis = _prep_call(x, W, degp)
    spart = _msg_kernel(row2, col2, ewf, hp)
    y = _post_call(spart, hp, dis, b_conv, batch32, gn_weight, gn_bias,
                   gn_mean_scale)
    return y
